# baseline (speedup 1.0000x reference)
import functools

import jax
import jax.numpy as jnp
from jax import lax
from jax.experimental import pallas as pl
from jax.experimental.pallas import tpu as pltpu
from jax.experimental.pallas import tpu_sc as plsc

_NC = 2
_NS = 16
_NW = _NC * _NS
_CB = 4
_NBUF = 4


@functools.lru_cache(maxsize=None)
def _make_gather(N, F, D):
    bpw = N // _NW
    n_chunks = bpw // _CB
    ch = _CB * F
    rpw = bpw * F
    assert n_chunks % _NBUF == 0 and ch <= 128
    mesh = plsc.VectorSubcoreMesh(
        core_axis_name="c", subcore_axis_name="s",
        num_cores=_NC, num_subcores=_NS)

    @functools.partial(
        pl.kernel,
        out_type=jax.ShapeDtypeStruct((N, F, D), jnp.float32),
        mesh=mesh,
        compiler_params=pltpu.CompilerParams(use_tc_tiling_on_sc=True),
        scratch_types=[
            pltpu.VMEM((rpw,), jnp.int32),
            [pltpu.VMEM((ch, D), jnp.float32)] * _NBUF,
            [pltpu.SemaphoreType.DMA] * _NBUF,
            [pltpu.SemaphoreType.DMA] * _NBUF,
        ],
    )
    def gather_kernel(table_hbm, idx_hbm, out_hbm, idx_v, bufs,
                      gsems, wsems):
        wid = lax.axis_index("s") * _NC + lax.axis_index("c")
        b_base = wid * bpw
        pltpu.sync_copy(idx_hbm.at[pl.ds(wid * rpw, rpw)], idx_v)

        def start_gather(c, b):
            pltpu.async_copy(
                table_hbm.at[idx_v.at[pl.ds(c * ch, ch)]], bufs[b], gsems[b])

        def wait_gather(b):
            pltpu.make_async_copy(
                table_hbm.at[idx_v.at[pl.ds(0, ch)]], bufs[b], gsems[b]).wait()

        def start_write(c, b):
            b0 = b_base + c * _CB
            for j in range(_CB):
                pltpu.async_copy(
                    bufs[b].at[pl.ds(j * F, F)], out_hbm.at[b0 + j],
                    wsems[b])

        def wait_write(b):
            for j in range(_CB):
                pltpu.make_async_copy(
                    bufs[b].at[pl.ds(0, F)], out_hbm.at[0], wsems[b]).wait()

        for b in range(_NBUF):
            start_gather(b, b)

        def loop_body(p, carry):
            for b in range(_NBUF):
                c = p * _NBUF + b
                wait_gather(b)
                start_write(c, b)
                nxt = c + _NBUF

                @pl.when(nxt < n_chunks)
                def _():
                    wait_write(b)
                    start_gather(nxt, b)
            return carry

        lax.fori_loop(0, n_chunks // _NBUF, loop_body, 0)
        for b in range(_NBUF):
            wait_write(b)

    return gather_kernel


def kernel(table, indices):
    N, F = indices.shape
    D = table.shape[1]
    return _make_gather(N, F, D)(table, indices.reshape(-1))


# speedup vs baseline: 1.8283x; 1.8283x over previous
import functools

import jax
import jax.numpy as jnp
from jax import lax
from jax.experimental import pallas as pl
from jax.experimental.pallas import tpu as pltpu
from jax.experimental.pallas import tpu_sc as plsc

_NC = 2
_NS = 16
_NW = _NC * _NS
_CH = 104
_NBUF = 4


@functools.lru_cache(maxsize=None)
def _make_gather(B, D):
    n_chunks = B // (_NW * _CH)
    assert n_chunks % _NBUF == 0
    mesh = plsc.VectorSubcoreMesh(
        core_axis_name="c", subcore_axis_name="s",
        num_cores=_NC, num_subcores=_NS)

    @functools.partial(
        pl.kernel,
        out_type=jax.ShapeDtypeStruct((B, D), jnp.float32),
        mesh=mesh,
        scratch_types=[
            pltpu.VMEM((n_chunks, _CH), jnp.int32),
            [pltpu.VMEM((_CH, D), jnp.float32)] * _NBUF,
            [pltpu.SemaphoreType.DMA] * _NBUF,
            [pltpu.SemaphoreType.DMA] * _NBUF,
        ],
    )
    def gather_kernel(table_hbm, idx_hbm, out_hbm, idx_v, bufs,
                      gsems, wsems):
        wid = lax.axis_index("s") * _NC + lax.axis_index("c")
        base = wid * (n_chunks * _CH)
        pltpu.sync_copy(idx_hbm.at[wid], idx_v)

        def start_gather(c, b):
            pltpu.async_copy(table_hbm.at[idx_v.at[c]], bufs[b], gsems[b])

        def wait_gather(b):
            pltpu.make_async_copy(
                table_hbm.at[idx_v.at[0]], bufs[b], gsems[b]).wait()

        def out_slice(c):
            return out_hbm.at[pl.ds(base + c * _CH, _CH)]

        def start_write(c, b):
            pltpu.async_copy(bufs[b], out_slice(c), wsems[b])

        def wait_write(b):
            pltpu.make_async_copy(bufs[b], out_slice(0), wsems[b]).wait()

        for b in range(_NBUF):
            start_gather(b, b)

        def loop_body(p, carry):
            for b in range(_NBUF):
                c = p * _NBUF + b
                wait_gather(b)
                start_write(c, b)
                nxt = c + _NBUF

                @pl.when(nxt < n_chunks)
                def _():
                    wait_write(b)
                    start_gather(nxt, b)
            return carry

        lax.fori_loop(0, n_chunks // _NBUF, loop_body, 0)
        for b in range(_NBUF):
            wait_write(b)

    return gather_kernel


def kernel(table, indices):
    N, F = indices.shape
    D = table.shape[1]
    B = N * F
    idx = indices.T.reshape(_NW, B // (_NW * _CH), _CH)
    out = _make_gather(B, D)(table, idx)
    return out.reshape(F, N, D).transpose(1, 0, 2)


# speedup vs baseline: 1.8460x; 1.0096x over previous
import functools

import jax
import jax.numpy as jnp
from jax import lax
from jax.experimental import pallas as pl
from jax.experimental.pallas import tpu as pltpu
from jax.experimental.pallas import tpu_sc as plsc

_NC = 2
_NS = 16
_NW = _NC * _NS
_CH = 104
_NBUF = 8


@functools.lru_cache(maxsize=None)
def _make_gather(B, D):
    n_chunks = B // (_NW * _CH)
    assert n_chunks % _NBUF == 0
    mesh = plsc.VectorSubcoreMesh(
        core_axis_name="c", subcore_axis_name="s",
        num_cores=_NC, num_subcores=_NS)

    @functools.partial(
        pl.kernel,
        out_type=jax.ShapeDtypeStruct((B, D), jnp.float32),
        mesh=mesh,
        scratch_types=[
            pltpu.VMEM((n_chunks, _CH), jnp.int32),
            [pltpu.VMEM((_CH, D), jnp.float32)] * _NBUF,
            [pltpu.SemaphoreType.DMA] * _NBUF,
            [pltpu.SemaphoreType.DMA] * _NBUF,
        ],
    )
    def gather_kernel(table_hbm, idx_hbm, out_hbm, idx_v, bufs,
                      gsems, wsems):
        wid = lax.axis_index("s") * _NC + lax.axis_index("c")
        base = wid * (n_chunks * _CH)
        pltpu.sync_copy(idx_hbm.at[wid], idx_v)

        def start_gather(c, b):
            pltpu.async_copy(table_hbm.at[idx_v.at[c]], bufs[b], gsems[b])

        def wait_gather(b):
            pltpu.make_async_copy(
                table_hbm.at[idx_v.at[0]], bufs[b], gsems[b]).wait()

        def out_slice(c):
            return out_hbm.at[pl.ds(base + c * _CH, _CH)]

        def start_write(c, b):
            pltpu.async_copy(bufs[b], out_slice(c), wsems[b])

        def wait_write(b):
            pltpu.make_async_copy(bufs[b], out_slice(0), wsems[b]).wait()

        for b in range(_NBUF):
            start_gather(b, b)

        def loop_body(p, carry):
            for b in range(_NBUF):
                c = p * _NBUF + b
                wait_gather(b)
                start_write(c, b)
                nxt = c + _NBUF

                @pl.when(nxt < n_chunks)
                def _():
                    wait_write(b)
                    start_gather(nxt, b)
            return carry

        lax.fori_loop(0, n_chunks // _NBUF, loop_body, 0)
        for b in range(_NBUF):
            wait_write(b)

    return gather_kernel


def kernel(table, indices):
    N, F = indices.shape
    D = table.shape[1]
    B = N * F
    idx = indices.T.reshape(_NW, B // (_NW * _CH), _CH)
    out = _make_gather(B, D)(table, idx)
    return out.reshape(F, N, D).transpose(1, 0, 2)


# speedup vs baseline: 1.8620x; 1.0087x over previous
import functools

import jax
import jax.numpy as jnp
from jax import lax
from jax.experimental import pallas as pl
from jax.experimental.pallas import tpu as pltpu
from jax.experimental.pallas import tpu_sc as plsc

_NC = 2
_NS = 16
_NW = _NC * _NS
_CH = 104
_NBUF = 8


@functools.lru_cache(maxsize=None)
def _make_gather(B, D):
    n_chunks = B // (_NW * _CH)
    assert n_chunks % _NBUF == 0
    mesh = plsc.VectorSubcoreMesh(
        core_axis_name="c", subcore_axis_name="s",
        num_cores=_NC, num_subcores=_NS)

    @functools.partial(
        pl.kernel,
        out_type=jax.ShapeDtypeStruct((B, D), jnp.float32),
        mesh=mesh,
        scratch_types=[
            pltpu.VMEM((n_chunks, _CH), jnp.int32),
            [pltpu.VMEM((_CH, D), jnp.float32)] * _NBUF,
            [pltpu.SemaphoreType.DMA] * _NBUF,
            [pltpu.SemaphoreType.DMA] * _NBUF,
        ],
    )
    def gather_kernel(table_hbm, idx_hbm, out_hbm, idx_v, bufs,
                      gsems, wsems):
        wid = lax.axis_index("s") * _NC + lax.axis_index("c")
        base = wid * (n_chunks * _CH)
        pltpu.sync_copy(idx_hbm.at[wid], idx_v)

        def start_gather(c, b):
            pltpu.async_copy(table_hbm.at[idx_v.at[c]], bufs[b], gsems[b])

        def wait_gather(b):
            pltpu.make_async_copy(
                table_hbm.at[idx_v.at[0]], bufs[b], gsems[b]).wait()

        def out_slice(c):
            return out_hbm.at[pl.ds(base + c * _CH, _CH)]

        def start_write(c, b):
            pltpu.async_copy(bufs[b], out_slice(c), wsems[b])

        def wait_write(b):
            pltpu.make_async_copy(bufs[b], out_slice(0), wsems[b]).wait()

        for b in range(_NBUF):
            start_gather(b, b)

        skew = _NBUF // 2

        def loop_body(p, carry):
            for b in range(_NBUF):
                c = p * _NBUF + b
                r = c - skew
                rb = (b - skew) % _NBUF

                @pl.when(jnp.logical_and(r >= 0, r + _NBUF < n_chunks))
                def _():
                    wait_write(rb)
                    start_gather(r + _NBUF, rb)

                wait_gather(b)
                start_write(c, b)
            return carry

        lax.fori_loop(0, n_chunks // _NBUF, loop_body, 0)
        for b in range(_NBUF):
            wait_write(b)

    return gather_kernel


def kernel(table, indices):
    N, F = indices.shape
    D = table.shape[1]
    B = N * F
    idx = indices.T.reshape(_NW, B // (_NW * _CH), _CH)
    out = _make_gather(B, D)(table, idx)
    return out.reshape(F, N, D).transpose(1, 0, 2)


# speedup vs baseline: 1.8682x; 1.0033x over previous
import functools

import jax
import jax.numpy as jnp
from jax import lax
from jax.experimental import pallas as pl
from jax.experimental.pallas import tpu as pltpu
from jax.experimental.pallas import tpu_sc as plsc

_NC = 2
_NS = 16
_NW = _NC * _NS
_CH = 104
_NBUF = 8


@functools.lru_cache(maxsize=None)
def _make_gather(B, D):
    n_chunks = B // (_NW * _CH)
    assert n_chunks % _NBUF == 0
    mesh = plsc.VectorSubcoreMesh(
        core_axis_name="c", subcore_axis_name="s",
        num_cores=_NC, num_subcores=_NS)

    @functools.partial(
        pl.kernel,
        out_type=jax.ShapeDtypeStruct((B, D), jnp.float32),
        mesh=mesh,
        compiler_params=pltpu.CompilerParams(
            disable_bounds_checks=True, skip_device_barrier=True),
        scratch_types=[
            pltpu.VMEM((n_chunks, _CH), jnp.int32),
            [pltpu.VMEM((_CH, D), jnp.float32)] * _NBUF,
            [pltpu.SemaphoreType.DMA] * _NBUF,
            [pltpu.SemaphoreType.DMA] * _NBUF,
        ],
    )
    def gather_kernel(table_hbm, idx_hbm, out_hbm, idx_v, bufs,
                      gsems, wsems):
        wid = lax.axis_index("s") * _NC + lax.axis_index("c")
        base = wid * (n_chunks * _CH)
        pltpu.sync_copy(idx_hbm.at[wid], idx_v)

        def start_gather(c, b):
            pltpu.async_copy(table_hbm.at[idx_v.at[c]], bufs[b], gsems[b])

        def wait_gather(b):
            pltpu.make_async_copy(
                table_hbm.at[idx_v.at[0]], bufs[b], gsems[b]).wait()

        def out_slice(c):
            return out_hbm.at[pl.ds(base + c * _CH, _CH)]

        def start_write(c, b):
            pltpu.async_copy(bufs[b], out_slice(c), wsems[b])

        def wait_write(b):
            pltpu.make_async_copy(bufs[b], out_slice(0), wsems[b]).wait()

        for b in range(_NBUF):
            start_gather(b, b)

        skew = _NBUF // 2

        def loop_body(p, carry):
            for b in range(_NBUF):
                c = p * _NBUF + b
                r = c - skew
                rb = (b - skew) % _NBUF

                @pl.when(jnp.logical_and(r >= 0, r + _NBUF < n_chunks))
                def _():
                    wait_write(rb)
                    start_gather(r + _NBUF, rb)

                wait_gather(b)
                start_write(c, b)
            return carry

        lax.fori_loop(0, n_chunks // _NBUF, loop_body, 0)
        for b in range(_NBUF):
            wait_write(b)

    return gather_kernel


def kernel(table, indices):
    N, F = indices.shape
    D = table.shape[1]
    B = N * F
    idx = indices.T.reshape(_NW, B // (_NW * _CH), _CH)
    out = _make_gather(B, D)(table, idx)
    return out.reshape(F, N, D).transpose(1, 0, 2)
